# baseline (speedup 1.0000x reference)
import functools

import jax
import jax.numpy as jnp
from jax import lax
from jax.experimental import pallas as pl
from jax.experimental.pallas import tpu as pltpu
from jax.experimental.pallas import tpu_sc as plsc

_N = 10000
_E = 320000
_D = 128
_NUM_LAYERS = 4

_NC = 2
_NS = 16
_NW = _NC * _NS
_G = 128
_GP = 80
_E_PAD = _NW * _GP * _G
_N_PAD = 10240
_RPT = _N_PAD // _NS
_CW = 128

_NCHUNK = 4
_GPC = (28, 12)
_GP0 = _NCHUNK * _GPC[0]
_GP1 = _NCHUNK * _GPC[1]
_CHUNK = 32
_GPT = _NCHUNK * _CHUNK


def _sc_segment_sum():
  mesh = plsc.VectorSubcoreMesh(core_axis_name="c", subcore_axis_name="s",
                                num_cores=_NC, num_subcores=_NS)

  out_type = [jax.ShapeDtypeStruct((_NC, _N_PAD, _D), jnp.float32)]
  scratch = [
      pltpu.VMEM((_CHUNK, _G), jnp.int32),
      pltpu.VMEM((_CHUNK, _G), jnp.int32),
      pltpu.VMEM((_G, _D), jnp.float32),
      pltpu.VMEM((_G, _D), jnp.float32),
      pltpu.VMEM_SHARED((_N_PAD, _D), jnp.float32),
      pltpu.SemaphoreType.DMA,
      pltpu.SemaphoreType.DMA,
  ]

  def body(h_hbm, src_hbm, dst_hbm, zeros_hbm, out_hbm,
           src_v, dst_v, rows_a, rows_b, acc_sh, sem_a, sem_b):
    c = lax.axis_index("c")
    s = lax.axis_index("s")
    wid = c * _NS + s
    cnt = jnp.where(c == 0, _GPC[0], _GPC[1])
    pairs = (cnt - 2) // 2

    pltpu.sync_copy(zeros_hbm.at[pl.ds(s * _RPT, _RPT)],
                    acc_sh.at[pl.ds(s * _RPT, _RPT)])
    plsc.subcore_barrier()

    for k in range(_NCHUNK):
      pltpu.sync_copy(src_hbm.at[wid, pl.ds(k * _CHUNK, _CHUNK)], src_v)
      pltpu.sync_copy(dst_hbm.at[wid, pl.ds(k * _CHUNK, _CHUNK)], dst_v)

      pltpu.async_copy(h_hbm.at[src_v.at[0]], rows_a, sem_a)
      pltpu.async_copy(h_hbm.at[src_v.at[1]], rows_b, sem_b)

      def step(i, carry):
        g = 2 * i

        def one(g, rows_v, sem):
          pltpu.make_async_copy(h_hbm.at[src_v.at[0]], rows_v, sem).wait()
          pltpu.sync_copy(rows_v, acc_sh.at[dst_v.at[g]], add=True)
          pltpu.async_copy(h_hbm.at[src_v.at[g + 2]], rows_v, sem)

        one(g, rows_a, sem_a)
        one(g + 1, rows_b, sem_b)
        return carry

      lax.fori_loop(0, pairs, step, 0)
      pltpu.make_async_copy(h_hbm.at[src_v.at[0]], rows_a, sem_a).wait()
      pltpu.sync_copy(rows_a, acc_sh.at[dst_v.at[cnt - 2]], add=True)
      pltpu.make_async_copy(h_hbm.at[src_v.at[0]], rows_b, sem_b).wait()
      pltpu.sync_copy(rows_b, acc_sh.at[dst_v.at[cnt - 1]], add=True)
    plsc.subcore_barrier()

    pltpu.sync_copy(acc_sh.at[pl.ds(s * _RPT, _RPT)],
                    out_hbm.at[c, pl.ds(s * _RPT, _RPT)])

  return pl.kernel(body, out_type=out_type, mesh=mesh, scratch_types=scratch)


def _sc_counts():
  mesh = plsc.VectorSubcoreMesh(core_axis_name="c", subcore_axis_name="s",
                                num_cores=_NC, num_subcores=_NS)
  out_type = [jax.ShapeDtypeStruct((_NC, _N_PAD, _CW), jnp.float32)]
  scratch = [
      pltpu.VMEM((_GP, _G), jnp.int32),
      pltpu.VMEM((_G, _CW), jnp.float32),
      pltpu.VMEM_SHARED((_N_PAD, _CW), jnp.float32),
  ]

  def body(dst_hbm, zc_hbm, ones_hbm, cnt_hbm, dst_v, ones_v, cnt_sh):
    c = lax.axis_index("c")
    s = lax.axis_index("s")
    wid = s * _NC + c

    pltpu.sync_copy(zc_hbm.at[pl.ds(s * _RPT, _RPT)],
                    cnt_sh.at[pl.ds(s * _RPT, _RPT)])
    pltpu.sync_copy(ones_hbm, ones_v)
    pltpu.sync_copy(dst_hbm.at[wid], dst_v)
    plsc.subcore_barrier()

    def step(g, carry):
      pltpu.sync_copy(ones_v, cnt_sh.at[dst_v.at[g]], add=True)
      return carry

    lax.fori_loop(0, _GP, step, 0)
    plsc.subcore_barrier()
    pltpu.sync_copy(cnt_sh.at[pl.ds(s * _RPT, _RPT)],
                    cnt_hbm.at[c, pl.ds(s * _RPT, _RPT)])

  return pl.kernel(body, out_type=out_type, mesh=mesh, scratch_types=scratch)


def _tc_layer(act: str, blk: int = 2048):
  def body(p0_ref, p1_ref, c0_ref, c1_ref, h_ref, wl_ref, wr_ref, b_ref, o_ref):
    cnt = c0_ref[:, 0:1] + c1_ref[:, 0:1]
    inv = 1.0 / jnp.maximum(cnt, 1.0)
    mean = (p0_ref[...] + p1_ref[...]) * inv
    acc = lax.dot_general(mean, wl_ref[...], (((1,), (1,)), ((), ())),
                          preferred_element_type=jnp.float32)
    acc += lax.dot_general(h_ref[...], wr_ref[...], (((1,), (1,)), ((), ())),
                           preferred_element_type=jnp.float32)
    acc += b_ref[...]
    if act == "relu":
      acc = jnp.maximum(acc, 0.0)
    elif act == "sigmoid":
      acc = jax.nn.sigmoid(acc)
    o_ref[...] = acc

  grid = (_N_PAD // blk,)
  row_spec = pl.BlockSpec((blk, _D), lambda i: (i, 0))
  cnt_spec = pl.BlockSpec((blk, _CW), lambda i: (i, 0))
  w_spec = pl.BlockSpec((_D, _D), lambda i: (0, 0))
  b_spec = pl.BlockSpec((1, _D), lambda i: (0, 0))
  return pl.pallas_call(
      body,
      grid=grid,
      in_specs=[row_spec, row_spec, cnt_spec, cnt_spec, row_spec,
                w_spec, w_spec, b_spec],
      out_specs=row_spec,
      out_shape=jax.ShapeDtypeStruct((_N_PAD, _D), jnp.float32),
  )


@jax.jit
def kernel(x, edge_index, Wl0, Wr0, b0, Wl1, Wr1, b1, Wl2, Wr2, b2, Wl3, Wr3, b3):
  src = edge_index[0]
  dst = edge_index[1]
  pad = _E_PAD - _E
  src_f = jnp.concatenate([src, jnp.zeros((pad,), jnp.int32)])
  dst_f = jnp.concatenate([dst, jnp.full((pad,), _N_PAD - 1, jnp.int32)])

  dst_u = dst_f.reshape(_NW, _GP, _G)

  ngroups0 = _NS * _GP0

  def skewed(flat, fill):
    g = flat.reshape(-1, _G)
    parts = []
    for c, gpc in enumerate(_GPC):
      lo = 0 if c == 0 else ngroups0
      blk = g[lo:lo + _NS * _NCHUNK * gpc].reshape(_NS, _NCHUNK, gpc, _G)
      blk = jnp.pad(blk, ((0, 0), (0, 0), (0, _CHUNK - gpc), (0, 0)),
                    constant_values=fill)
      parts.append(blk.reshape(_NS, _GPT, _G))
    return jnp.concatenate(parts, axis=0)

  src_p = skewed(src_f, 0)
  dst_p = skewed(dst_f, _N_PAD - 1)

  h = jnp.zeros((_N_PAD, _D), jnp.float32).at[:_N].set(x)
  zeros = jnp.zeros((_N_PAD, _D), jnp.float32)
  zeros_c = jnp.zeros((_N_PAD, _CW), jnp.float32)
  ones = jnp.ones((_G, _CW), jnp.float32)

  sc_sum = _sc_segment_sum()
  (cnt,) = _sc_counts()(dst_u, zeros_c, ones)

  layers = [(Wl0, Wr0, b0), (Wl1, Wr1, b1), (Wl2, Wr2, b2), (Wl3, Wr3, b3)]
  for i, (Wl, Wr, b) in enumerate(layers):
    (partial,) = sc_sum(h, src_p, dst_p, zeros)
    act = "relu" if i < _NUM_LAYERS - 1 else "sigmoid"
    h = _tc_layer(act)(partial[0], partial[1], cnt[0], cnt[1], h,
                       Wl, Wr, b.reshape(1, _D))
  return h[:_N]


# speedup vs baseline: 1.1689x; 1.1689x over previous
import functools

import jax
import jax.numpy as jnp
from jax import lax
from jax.experimental import pallas as pl
from jax.experimental.pallas import tpu as pltpu
from jax.experimental.pallas import tpu_sc as plsc

_N = 10000
_E = 320000
_D = 128
_NUM_LAYERS = 4

_NC = 2
_NS = 16
_NW = _NC * _NS
_G = 128
_GP = 80
_E_PAD = _NW * _GP * _G
_N_PAD = 10240
_RPT = _N_PAD // _NS
_CW = 128

_NCHUNK = 4
_GPC = (32, 8)
_GP0 = _NCHUNK * _GPC[0]
_GP1 = _NCHUNK * _GPC[1]
_CHUNK = 32
_GPT = _NCHUNK * _CHUNK


def _sc_segment_sum():
  mesh = plsc.VectorSubcoreMesh(core_axis_name="c", subcore_axis_name="s",
                                num_cores=_NC, num_subcores=_NS)

  out_type = [jax.ShapeDtypeStruct((_NC, _N_PAD, _D), jnp.float32)]
  scratch = [
      pltpu.VMEM((_CHUNK, _G), jnp.int32),
      pltpu.VMEM((_CHUNK, _G), jnp.int32),
      pltpu.VMEM((_G, _D), jnp.float32),
      pltpu.VMEM((_G, _D), jnp.float32),
      pltpu.VMEM_SHARED((_N_PAD, _D), jnp.float32),
      pltpu.SemaphoreType.DMA,
      pltpu.SemaphoreType.DMA,
  ]

  def body(h_hbm, src_hbm, dst_hbm, zeros_hbm, out_hbm,
           src_v, dst_v, rows_a, rows_b, acc_sh, sem_a, sem_b):
    c = lax.axis_index("c")
    s = lax.axis_index("s")
    wid = c * _NS + s
    cnt = jnp.where(c == 0, _GPC[0], _GPC[1])
    pairs = (cnt - 2) // 2

    pltpu.sync_copy(zeros_hbm.at[pl.ds(s * _RPT, _RPT)],
                    acc_sh.at[pl.ds(s * _RPT, _RPT)])
    plsc.subcore_barrier()

    for k in range(_NCHUNK):
      pltpu.sync_copy(src_hbm.at[wid, pl.ds(k * _CHUNK, _CHUNK)], src_v)
      pltpu.sync_copy(dst_hbm.at[wid, pl.ds(k * _CHUNK, _CHUNK)], dst_v)

      pltpu.async_copy(h_hbm.at[src_v.at[0]], rows_a, sem_a)
      pltpu.async_copy(h_hbm.at[src_v.at[1]], rows_b, sem_b)

      def step(i, carry):
        g = 2 * i

        def one(g, rows_v, sem):
          pltpu.make_async_copy(h_hbm.at[src_v.at[0]], rows_v, sem).wait()
          pltpu.sync_copy(rows_v, acc_sh.at[dst_v.at[g]], add=True)
          pltpu.async_copy(h_hbm.at[src_v.at[g + 2]], rows_v, sem)

        one(g, rows_a, sem_a)
        one(g + 1, rows_b, sem_b)
        return carry

      lax.fori_loop(0, pairs, step, 0)
      pltpu.make_async_copy(h_hbm.at[src_v.at[0]], rows_a, sem_a).wait()
      pltpu.sync_copy(rows_a, acc_sh.at[dst_v.at[cnt - 2]], add=True)
      pltpu.make_async_copy(h_hbm.at[src_v.at[0]], rows_b, sem_b).wait()
      pltpu.sync_copy(rows_b, acc_sh.at[dst_v.at[cnt - 1]], add=True)
    plsc.subcore_barrier()

    pltpu.sync_copy(acc_sh.at[pl.ds(s * _RPT, _RPT)],
                    out_hbm.at[c, pl.ds(s * _RPT, _RPT)])

  return pl.kernel(body, out_type=out_type, mesh=mesh, scratch_types=scratch)


def _sc_counts():
  mesh = plsc.VectorSubcoreMesh(core_axis_name="c", subcore_axis_name="s",
                                num_cores=_NC, num_subcores=_NS)
  out_type = [jax.ShapeDtypeStruct((_NC, _N_PAD, _CW), jnp.float32)]
  scratch = [
      pltpu.VMEM((_GP, _G), jnp.int32),
      pltpu.VMEM((_G, _CW), jnp.float32),
      pltpu.VMEM_SHARED((_N_PAD, _CW), jnp.float32),
  ]

  def body(dst_hbm, zc_hbm, ones_hbm, cnt_hbm, dst_v, ones_v, cnt_sh):
    c = lax.axis_index("c")
    s = lax.axis_index("s")
    wid = s * _NC + c

    pltpu.sync_copy(zc_hbm.at[pl.ds(s * _RPT, _RPT)],
                    cnt_sh.at[pl.ds(s * _RPT, _RPT)])
    pltpu.sync_copy(ones_hbm, ones_v)
    pltpu.sync_copy(dst_hbm.at[wid], dst_v)
    plsc.subcore_barrier()

    def step(g, carry):
      pltpu.sync_copy(ones_v, cnt_sh.at[dst_v.at[g]], add=True)
      return carry

    lax.fori_loop(0, _GP, step, 0)
    plsc.subcore_barrier()
    pltpu.sync_copy(cnt_sh.at[pl.ds(s * _RPT, _RPT)],
                    cnt_hbm.at[c, pl.ds(s * _RPT, _RPT)])

  return pl.kernel(body, out_type=out_type, mesh=mesh, scratch_types=scratch)


def _tc_layer(act: str, blk: int = 2048):
  def body(p0_ref, p1_ref, c0_ref, c1_ref, h_ref, wl_ref, wr_ref, b_ref, o_ref):
    cnt = c0_ref[:, 0:1] + c1_ref[:, 0:1]
    inv = 1.0 / jnp.maximum(cnt, 1.0)
    mean = (p0_ref[...] + p1_ref[...]) * inv
    acc = lax.dot_general(mean, wl_ref[...], (((1,), (1,)), ((), ())),
                          preferred_element_type=jnp.float32)
    acc += lax.dot_general(h_ref[...], wr_ref[...], (((1,), (1,)), ((), ())),
                           preferred_element_type=jnp.float32)
    acc += b_ref[...]
    if act == "relu":
      acc = jnp.maximum(acc, 0.0)
    elif act == "sigmoid":
      acc = jax.nn.sigmoid(acc)
    o_ref[...] = acc

  grid = (_N_PAD // blk,)
  row_spec = pl.BlockSpec((blk, _D), lambda i: (i, 0))
  cnt_spec = pl.BlockSpec((blk, _CW), lambda i: (i, 0))
  w_spec = pl.BlockSpec((_D, _D), lambda i: (0, 0))
  b_spec = pl.BlockSpec((1, _D), lambda i: (0, 0))
  return pl.pallas_call(
      body,
      grid=grid,
      in_specs=[row_spec, row_spec, cnt_spec, cnt_spec, row_spec,
                w_spec, w_spec, b_spec],
      out_specs=row_spec,
      out_shape=jax.ShapeDtypeStruct((_N_PAD, _D), jnp.float32),
  )


@jax.jit
def kernel(x, edge_index, Wl0, Wr0, b0, Wl1, Wr1, b1, Wl2, Wr2, b2, Wl3, Wr3, b3):
  src = edge_index[0]
  dst = edge_index[1]
  pad = _E_PAD - _E
  src_f = jnp.concatenate([src, jnp.zeros((pad,), jnp.int32)])
  dst_f = jnp.concatenate([dst, jnp.full((pad,), _N_PAD - 1, jnp.int32)])

  dst_u = dst_f.reshape(_NW, _GP, _G)

  ngroups0 = _NS * _GP0

  def skewed(flat, fill):
    g = flat.reshape(-1, _G)
    parts = []
    for c, gpc in enumerate(_GPC):
      lo = 0 if c == 0 else ngroups0
      blk = g[lo:lo + _NS * _NCHUNK * gpc].reshape(_NS, _NCHUNK, gpc, _G)
      blk = jnp.pad(blk, ((0, 0), (0, 0), (0, _CHUNK - gpc), (0, 0)),
                    constant_values=fill)
      parts.append(blk.reshape(_NS, _GPT, _G))
    return jnp.concatenate(parts, axis=0)

  src_p = skewed(src_f, 0)
  dst_p = skewed(dst_f, _N_PAD - 1)

  h = jnp.zeros((_N_PAD, _D), jnp.float32).at[:_N].set(x)
  zeros = jnp.zeros((_N_PAD, _D), jnp.float32)
  zeros_c = jnp.zeros((_N_PAD, _CW), jnp.float32)
  ones = jnp.ones((_G, _CW), jnp.float32)

  sc_sum = _sc_segment_sum()
  (cnt,) = _sc_counts()(dst_u, zeros_c, ones)

  layers = [(Wl0, Wr0, b0), (Wl1, Wr1, b1), (Wl2, Wr2, b2), (Wl3, Wr3, b3)]
  for i, (Wl, Wr, b) in enumerate(layers):
    (partial,) = sc_sum(h, src_p, dst_p, zeros)
    act = "relu" if i < _NUM_LAYERS - 1 else "sigmoid"
    h = _tc_layer(act)(partial[0], partial[1], cnt[0], cnt[1], h,
                       Wl, Wr, b.reshape(1, _D))
  return h[:_N]
